# baseline (speedup 1.0000x reference)
import jax
import jax.numpy as jnp
from jax import lax
from jax.experimental import pallas as pl
from jax.experimental.pallas import tpu as pltpu
from jax.experimental.pallas import tpu_sc as plsc

VOCAB = 1000000
SEQ = 200
DIM = 64
BATCH = 1024
EPS = 1e-05

NC = 2
NS = 16
NW = NC * NS
L = 16

SPW = BATCH // NW
HOFF = (0, 104)
HLEN = (104, 96)

_GATHER_DNUMS = lax.GatherDimensionNumbers(
    offset_dims=(), collapsed_slice_dims=(0,), start_index_map=(0,))


def _shuffle(x, idx):
    return lax.gather(x, idx[:, None], _GATHER_DNUMS, (1,),
                      mode=lax.GatherScatterMode.PROMISE_IN_BOUNDS)


def _hsum(x):
    for sh in (8, 4, 2, 1):
        idx = lax.iota(jnp.int32, L) ^ sh
        x = x + _shuffle(x, idx)
    return x


def _rsqrt(x):
    i = lax.bitcast_convert_type(x, jnp.int32)
    i = jnp.full((L,), 0x5F3759DF, jnp.int32) - lax.shift_right_arithmetic(
        i, jnp.full((L,), 1, jnp.int32))
    y = lax.bitcast_convert_type(i, jnp.float32)
    y = y * (1.5 - 0.5 * x * y * y)
    y = y * (1.5 - 0.5 * x * y * y)
    return y


def _body(state_hbm, token_hbm, pos_hbm, gb_hbm, out_hbm,
          idx_v, rows_v, out_v, pos_v, gb_v,
          isem0, isem1, gsem00, gsem01, gsem10, gsem11, osem0, osem1):
    wid = lax.axis_index("s") * NC + lax.axis_index("c")
    base = wid * SPW

    pltpu.sync_copy(pos_hbm, pos_v)
    pltpu.sync_copy(gb_hbm, gb_v)

    g_vec = [gb_v[pl.ds(k * L, L)] for k in range(4)]
    b_vec = [gb_v[pl.ds(DIM + k * L, L)] for k in range(4)]
    isems = [isem0, isem1]
    gsems = [[gsem00, gsem01], [gsem10, gsem11]]
    osems = [osem0, osem1]

    def fetch_idx(c, p):
        pltpu.async_copy(state_hbm.at[pl.ds((base + c) * SEQ, SEQ)],
                         idx_v.at[p], isems[p])

    def drain_idx(p):
        pltpu.make_async_copy(state_hbm.at[pl.ds(0, SEQ)],
                              idx_v.at[p], isems[p]).wait()

    def fire_half(p, h):
        pltpu.async_copy(
            token_hbm.at[idx_v.at[p, pl.ds(HOFF[h], HLEN[h])]],
            rows_v.at[p, pl.ds(HOFF[h], HLEN[h])], gsems[p][h])

    def drain_half(p, h):
        pltpu.make_async_copy(token_hbm.at[pl.ds(0, HLEN[h])],
                              rows_v.at[p, pl.ds(HOFF[h], HLEN[h])],
                              gsems[p][h]).wait()

    def drain_out(p):
        pltpu.make_async_copy(out_v.at[p], out_hbm.at[base], osems[p]).wait()

    def compute_half(p, h):
        def row_step(r0, carry):
            r = HOFF[h] + r0
            x = [rows_v[p, r, pl.ds(k * L, L)]
                 + pos_v[pl.ds(r * DIM + k * L, L)] for k in range(4)]
            tot = _hsum((x[0] + x[1]) + (x[2] + x[3]))
            qtot = _hsum((x[0] * x[0] + x[1] * x[1])
                         + (x[2] * x[2] + x[3] * x[3]))
            mean = tot * (1.0 / DIM)
            var = qtot * (1.0 / DIM) - mean * mean
            rstd = _rsqrt(var + EPS)
            for k in range(4):
                out_v[p, r, pl.ds(k * L, L)] = ((x[k] - mean) * rstd
                                                * g_vec[k] + b_vec[k])
            return carry
        lax.fori_loop(0, HLEN[h], row_step, 0, unroll=2)

    fetch_idx(0, 0)
    drain_idx(0)
    fire_half(0, 0)
    fire_half(0, 1)
    fetch_idx(1, 1)

    def seq_body(c, p):
        @pl.when(c >= 2)
        def _():
            drain_out(p)

        @pl.when(c + 1 < SPW)
        def _():
            drain_idx(1 - p)
            fire_half(1 - p, 0)
            fire_half(1 - p, 1)

        drain_half(p, 0)
        compute_half(p, 0)
        drain_half(p, 1)

        @pl.when(c + 2 < SPW)
        def _():
            fetch_idx(c + 2, p)

        compute_half(p, 1)
        pltpu.async_copy(out_v.at[p], out_hbm.at[base + c], osems[p])

    def pair_step(half_c, carry):
        seq_body(half_c * 2, 0)
        seq_body(half_c * 2 + 1, 1)
        return carry

    lax.fori_loop(0, SPW // 2, pair_step, 0)
    drain_out(0)
    drain_out(1)


@jax.jit
def _run(state, token_table, pos_table, gb):
    mesh = plsc.VectorSubcoreMesh(core_axis_name="c", subcore_axis_name="s",
                                  num_cores=NC, num_subcores=NS)
    f = pl.kernel(
        _body,
        out_type=jax.ShapeDtypeStruct((BATCH, SEQ, DIM), jnp.float32),
        mesh=mesh,
        scratch_types=[
            pltpu.VMEM((2, SEQ), jnp.int32),
            pltpu.VMEM((2, SEQ, DIM), jnp.float32),
            pltpu.VMEM((2, SEQ, DIM), jnp.float32),
            pltpu.VMEM((SEQ * DIM,), jnp.float32),
            pltpu.VMEM((2 * DIM,), jnp.float32),
            pltpu.SemaphoreType.DMA,
            pltpu.SemaphoreType.DMA,
            pltpu.SemaphoreType.DMA,
            pltpu.SemaphoreType.DMA,
            pltpu.SemaphoreType.DMA,
            pltpu.SemaphoreType.DMA,
            pltpu.SemaphoreType.DMA,
            pltpu.SemaphoreType.DMA,
        ],
        compiler_params=pltpu.CompilerParams(use_tc_tiling_on_sc=False),
    )
    return f(state, token_table, pos_table, gb)


def kernel(state, token_table, pos_table, gamma, beta):
    gb = jnp.concatenate([gamma, beta])
    return _run(state.reshape(-1).astype(jnp.int32), token_table,
                pos_table.reshape(-1), gb)
